# speedup vs baseline: 2.0134x; 2.0134x over previous
import functools

import jax
import jax.numpy as jnp
from jax import lax
from jax.experimental import pallas as pl
from jax.experimental.pallas import tpu as pltpu
from jax.experimental.pallas import tpu_sc as plsc

_HIDDEN = 768
_EPS = 1e-12
_NC = 2
_NS = 16
_NW = _NC * _NS
_CHUNK = 64


def _sc_gather(word_table, idx_flat):
    n_tok = idx_flat.shape[0]
    per_w = n_tok // _NW
    n_ch = per_w // _CHUNK
    mesh = plsc.VectorSubcoreMesh(core_axis_name="c", subcore_axis_name="s")

    @functools.partial(
        pl.kernel,
        out_type=jax.ShapeDtypeStruct((n_tok, _HIDDEN), jnp.float32),
        mesh=mesh,
        scratch_types=[
            pltpu.VMEM((per_w,), jnp.int32),
            pltpu.VMEM((_CHUNK, _HIDDEN), jnp.float32),
            pltpu.SemaphoreType.DMA,
        ],
    )
    def gather_kernel(table_hbm, idx_hbm, out_hbm, idx_v, rows_v, sem):
        wid = lax.axis_index("s") * _NC + lax.axis_index("c")
        base = wid * per_w
        pltpu.sync_copy(idx_hbm.at[pl.ds(base, per_w)], idx_v)

        @pl.loop(0, n_ch)
        def _(c):
            off = c * _CHUNK
            pltpu.async_copy(
                table_hbm.at[idx_v.at[pl.ds(off, _CHUNK)]], rows_v, sem
            ).wait()
            pltpu.sync_copy(rows_v, out_hbm.at[pl.ds(base + off, _CHUNK)])

    return gather_kernel(word_table, idx_flat)


def _ln_body(g_ref, p_ref, tt_ref, ty_ref, w_ref, b_ref, o_ref):
    x = g_ref[...] + p_ref[...]
    t0 = ty_ref[0:1, :]
    t1 = ty_ref[1:2, :]
    x = x + t0 + tt_ref[...] * (t1 - t0)
    mean = jnp.mean(x, axis=1, keepdims=True)
    xc = x - mean
    var = jnp.mean(xc * xc, axis=1, keepdims=True)
    y = xc * lax.rsqrt(var + _EPS)
    o_ref[...] = y * w_ref[...] + b_ref[...]


def _tc_add_ln(gathered, pos_table, ttf, type_table, ln_w, ln_b, seq_len):
    n_tok = gathered.shape[0]
    t_blk = 256
    blocks_per_seq = seq_len // t_blk
    grid = (n_tok // t_blk,)
    return pl.pallas_call(
        _ln_body,
        grid=grid,
        in_specs=[
            pl.BlockSpec((t_blk, _HIDDEN), lambda i: (i, 0)),
            pl.BlockSpec((t_blk, _HIDDEN), lambda i: (i % blocks_per_seq, 0)),
            pl.BlockSpec((t_blk, 1), lambda i: (i, 0)),
            pl.BlockSpec((2, _HIDDEN), lambda i: (0, 0)),
            pl.BlockSpec((1, _HIDDEN), lambda i: (0, 0)),
            pl.BlockSpec((1, _HIDDEN), lambda i: (0, 0)),
        ],
        out_specs=pl.BlockSpec((t_blk, _HIDDEN), lambda i: (i, 0)),
        out_shape=jax.ShapeDtypeStruct((n_tok, _HIDDEN), jnp.float32),
    )(gathered, pos_table, ttf, type_table, ln_w, ln_b)


def kernel(input_ids, token_type_ids, word_table, pos_table, type_table, ln_weight, ln_bias):
    b, s = input_ids.shape
    idx = input_ids.reshape(-1).astype(jnp.int32)
    gathered = _sc_gather(word_table, idx)
    ttf = token_type_ids.reshape(-1, 1).astype(jnp.float32)
    out = _tc_add_ln(
        gathered,
        pos_table,
        ttf,
        type_table,
        ln_weight.reshape(1, -1),
        ln_bias.reshape(1, -1),
        s,
    )
    return out.reshape(b, s, _HIDDEN)


# speedup vs baseline: 2.0757x; 1.0309x over previous
import functools

import jax
import jax.numpy as jnp
from jax import lax
from jax.experimental import pallas as pl
from jax.experimental.pallas import tpu as pltpu
from jax.experimental.pallas import tpu_sc as plsc

_HIDDEN = 768
_EPS = 1e-12
_NC = 2
_NS = 16
_NW = _NC * _NS
_CHUNK = 64


def _sc_gather(word_table, idx_flat):
    n_tok = idx_flat.shape[0]
    per_w = n_tok // _NW
    n_ch = per_w // _CHUNK
    mesh = plsc.VectorSubcoreMesh(core_axis_name="c", subcore_axis_name="s")

    @functools.partial(
        pl.kernel,
        out_type=jax.ShapeDtypeStruct((n_tok, _HIDDEN), jnp.float32),
        mesh=mesh,
        scratch_types=[
            pltpu.VMEM((per_w,), jnp.int32),
            pltpu.VMEM((_CHUNK, _HIDDEN), jnp.float32),
            pltpu.VMEM((_CHUNK, _HIDDEN), jnp.float32),
            pltpu.SemaphoreType.DMA,
            pltpu.SemaphoreType.DMA,
        ],
    )
    def gather_kernel(table_hbm, idx_hbm, out_hbm, idx_v, rows_a, rows_b, sem_a, sem_b):
        wid = lax.axis_index("s") * _NC + lax.axis_index("c")
        base = wid * per_w
        pltpu.sync_copy(idx_hbm.at[pl.ds(base, per_w)], idx_v)

        bufs = (rows_a, rows_b)
        sems = (sem_a, sem_b)
        copies = [None] * n_ch
        copies[0] = pltpu.async_copy(
            table_hbm.at[idx_v.at[pl.ds(0, _CHUNK)]], bufs[0], sems[0]
        )
        for c in range(n_ch):
            if c + 1 < n_ch:
                copies[c + 1] = pltpu.async_copy(
                    table_hbm.at[idx_v.at[pl.ds((c + 1) * _CHUNK, _CHUNK)]],
                    bufs[(c + 1) % 2],
                    sems[(c + 1) % 2],
                )
            copies[c].wait()
            pltpu.sync_copy(bufs[c % 2], out_hbm.at[pl.ds(base + c * _CHUNK, _CHUNK)])

    return gather_kernel(word_table, idx_flat)


def _ln_body(g_ref, p_ref, tt_ref, ty_ref, w_ref, b_ref, o_ref):
    x = g_ref[0] + p_ref[...]
    t0 = ty_ref[0:1, :]
    t1 = ty_ref[1:2, :]
    tt = tt_ref[0].astype(jnp.float32)
    x = x + t0 + tt * (t1 - t0)
    mean = jnp.mean(x, axis=1, keepdims=True)
    xc = x - mean
    var = jnp.mean(xc * xc, axis=1, keepdims=True)
    y = xc * lax.rsqrt(var + _EPS)
    o_ref[0] = y * w_ref[...] + b_ref[...]


def _tc_add_ln(gathered3, pos_table, tt3, type_table, ln_w, ln_b):
    b, s, _ = gathered3.shape
    t_blk = 256
    grid = (s // t_blk, b)
    return pl.pallas_call(
        _ln_body,
        grid=grid,
        in_specs=[
            pl.BlockSpec((1, t_blk, _HIDDEN), lambda i, j: (j, i, 0)),
            pl.BlockSpec((t_blk, _HIDDEN), lambda i, j: (i, 0)),
            pl.BlockSpec((1, t_blk, 1), lambda i, j: (j, i, 0)),
            pl.BlockSpec((2, _HIDDEN), lambda i, j: (0, 0)),
            pl.BlockSpec((1, _HIDDEN), lambda i, j: (0, 0)),
            pl.BlockSpec((1, _HIDDEN), lambda i, j: (0, 0)),
        ],
        out_specs=pl.BlockSpec((1, t_blk, _HIDDEN), lambda i, j: (j, i, 0)),
        out_shape=jax.ShapeDtypeStruct((b, s, _HIDDEN), jnp.float32),
    )(gathered3, pos_table, tt3, type_table, ln_w, ln_b)


def kernel(input_ids, token_type_ids, word_table, pos_table, type_table, ln_weight, ln_bias):
    b, s = input_ids.shape
    idx = input_ids.reshape(-1).astype(jnp.int32)
    gathered = _sc_gather(word_table, idx).reshape(b, s, _HIDDEN)
    tt3 = token_type_ids.reshape(b, s, 1)
    return _tc_add_ln(
        gathered,
        pos_table,
        tt3,
        type_table,
        ln_weight.reshape(1, -1),
        ln_bias.reshape(1, -1),
    )


# speedup vs baseline: 2.1210x; 1.0218x over previous
import functools

import jax
import jax.numpy as jnp
from jax import lax
from jax.experimental import pallas as pl
from jax.experimental.pallas import tpu as pltpu
from jax.experimental.pallas import tpu_sc as plsc

_HIDDEN = 768
_EPS = 1e-12
_NC = 2
_NS = 16
_NW = _NC * _NS
_CHUNK = 64
_NCHUNK = 4
_TBLK = 256


def _sc_gather(word_table, idx_flat):
    n_tok = idx_flat.shape[0]
    per_w = n_tok // _NW
    n_ch = per_w // _CHUNK
    mesh = plsc.VectorSubcoreMesh(core_axis_name="c", subcore_axis_name="s")

    @functools.partial(
        pl.kernel,
        out_type=jax.ShapeDtypeStruct((n_tok, _HIDDEN), jnp.float32),
        mesh=mesh,
        scratch_types=[
            pltpu.VMEM((per_w,), jnp.int32),
            pltpu.VMEM((_CHUNK, _HIDDEN), jnp.float32),
            pltpu.VMEM((_CHUNK, _HIDDEN), jnp.float32),
            pltpu.SemaphoreType.DMA,
            pltpu.SemaphoreType.DMA,
        ],
    )
    def gather_kernel(table_hbm, idx_hbm, out_hbm, idx_v, rows_a, rows_b, sem_a, sem_b):
        wid = lax.axis_index("s") * _NC + lax.axis_index("c")
        base = wid * per_w
        pltpu.sync_copy(idx_hbm.at[pl.ds(base, per_w)], idx_v)

        bufs = (rows_a, rows_b)
        sems = (sem_a, sem_b)
        copies = [None] * n_ch
        copies[0] = pltpu.async_copy(
            table_hbm.at[idx_v.at[pl.ds(0, _CHUNK)]], bufs[0], sems[0]
        )
        for c in range(n_ch):
            if c + 1 < n_ch:
                copies[c + 1] = pltpu.async_copy(
                    table_hbm.at[idx_v.at[pl.ds((c + 1) * _CHUNK, _CHUNK)]],
                    bufs[(c + 1) % 2],
                    sems[(c + 1) % 2],
                )
            copies[c].wait()
            pltpu.sync_copy(bufs[c % 2], out_hbm.at[pl.ds(base + c * _CHUNK, _CHUNK)])

    return gather_kernel(word_table, idx_flat)


def _ln_body(g_ref, p_ref, tt_ref, ty_ref, w_ref, b_ref, o_ref):
    x = g_ref[0] + p_ref[...]
    t0 = ty_ref[0:1, :]
    t1 = ty_ref[1:2, :]
    tt = tt_ref[0].astype(jnp.float32)
    x = x + t0 + tt * (t1 - t0)
    mean = jnp.mean(x, axis=1, keepdims=True)
    xc = x - mean
    var = jnp.mean(xc * xc, axis=1, keepdims=True)
    y = xc * lax.rsqrt(var + _EPS)
    o_ref[0] = y * w_ref[...] + b_ref[...]


def _ln_body_acc(g_ref, p_ref, tt_ref, ty_ref, w_ref, b_ref, _buf_ref, o_ref):
    _ln_body(g_ref, p_ref, tt_ref, ty_ref, w_ref, b_ref, o_ref)


def _tc_add_ln_chunk(g_c, pos_table, tt_c, type_table, ln_w, ln_b, buf, c, b, s):
    s_ch = g_c.shape[1]
    sblk0 = c * (s_ch // _TBLK)
    grid = (s_ch // _TBLK, b)
    in_specs = [
        pl.BlockSpec((1, _TBLK, _HIDDEN), lambda i, j: (j, i, 0)),
        pl.BlockSpec((_TBLK, _HIDDEN), lambda i, j, c=sblk0: (c + i, 0)),
        pl.BlockSpec((1, _TBLK, 1), lambda i, j: (j, i, 0)),
        pl.BlockSpec((2, _HIDDEN), lambda i, j: (0, 0)),
        pl.BlockSpec((1, _HIDDEN), lambda i, j: (0, 0)),
        pl.BlockSpec((1, _HIDDEN), lambda i, j: (0, 0)),
    ]
    args = [g_c, pos_table, tt_c, type_table, ln_w, ln_b]
    body = _ln_body
    aliases = {}
    if buf is not None:
        in_specs.append(pl.BlockSpec(memory_space=pltpu.MemorySpace.HBM))
        args.append(buf)
        aliases = {6: 0}
        body = _ln_body_acc
    return pl.pallas_call(
        body,
        grid=grid,
        in_specs=in_specs,
        out_specs=pl.BlockSpec((1, _TBLK, _HIDDEN), lambda i, j, c=sblk0: (j, c + i, 0)),
        out_shape=jax.ShapeDtypeStruct((b, s, _HIDDEN), jnp.float32),
        input_output_aliases=aliases,
    )(*args)


def kernel(input_ids, token_type_ids, word_table, pos_table, type_table, ln_weight, ln_bias):
    b, s = input_ids.shape
    s_ch = s // _NCHUNK
    ln_w = ln_weight.reshape(1, -1)
    ln_b = ln_bias.reshape(1, -1)

    gathered = []
    for c in range(_NCHUNK):
        idx_c = input_ids[:, c * s_ch:(c + 1) * s_ch].reshape(-1).astype(jnp.int32)
        gathered.append(_sc_gather(word_table, idx_c).reshape(b, s_ch, _HIDDEN))

    buf = None
    for c in range(_NCHUNK):
        tt_c = token_type_ids[:, c * s_ch:(c + 1) * s_ch].reshape(b, s_ch, 1)
        buf = _tc_add_ln_chunk(
            gathered[c], pos_table, tt_c, type_table, ln_w, ln_b, buf, c, b, s
        )
    return buf


# speedup vs baseline: 2.2156x; 1.0446x over previous
import functools

import jax
import jax.numpy as jnp
from jax import lax
from jax.experimental import pallas as pl
from jax.experimental.pallas import tpu as pltpu
from jax.experimental.pallas import tpu_sc as plsc

_HIDDEN = 768
_EPS = 1e-12
_NC = 2
_NS = 16
_NW = _NC * _NS
_CHUNK = 32
_NCHUNK = 4
_TBLK = 256


def _sc_gather_chunk(word_table, idx_flat, chunk_base, n_rows, s, s_ch):
    per_w = n_rows // _NW
    n_sub = per_w // _CHUNK
    w_per_b = s_ch // per_w
    mesh = plsc.VectorSubcoreMesh(core_axis_name="c", subcore_axis_name="s")

    @functools.partial(
        pl.kernel,
        out_type=jax.ShapeDtypeStruct((n_rows, _HIDDEN), jnp.float32),
        mesh=mesh,
        scratch_types=[
            pltpu.VMEM((per_w,), jnp.int32),
            pltpu.VMEM((_CHUNK, _HIDDEN), jnp.float32),
            pltpu.VMEM((_CHUNK, _HIDDEN), jnp.float32),
            pltpu.SemaphoreType.DMA,
            pltpu.SemaphoreType.DMA,
        ],
    )
    def gather_kernel(table_hbm, idx_hbm, out_hbm, idx_v, rows_a, rows_b, sem_a, sem_b):
        wid = lax.axis_index("s") * _NC + lax.axis_index("c")
        b = wid // w_per_b
        j = (wid % w_per_b) * per_w
        src = b * s + chunk_base + j
        pltpu.sync_copy(idx_hbm.at[pl.ds(src, per_w)], idx_v)

        base = wid * per_w
        bufs = (rows_a, rows_b)
        sems = (sem_a, sem_b)
        copies = [None] * n_sub
        copies[0] = pltpu.async_copy(
            table_hbm.at[idx_v.at[pl.ds(0, _CHUNK)]], bufs[0], sems[0]
        )
        for c in range(n_sub):
            if c + 1 < n_sub:
                copies[c + 1] = pltpu.async_copy(
                    table_hbm.at[idx_v.at[pl.ds((c + 1) * _CHUNK, _CHUNK)]],
                    bufs[(c + 1) % 2],
                    sems[(c + 1) % 2],
                )
            copies[c].wait()
            pltpu.sync_copy(bufs[c % 2], out_hbm.at[pl.ds(base + c * _CHUNK, _CHUNK)])

    return gather_kernel(word_table, idx_flat)


def _ln_body(g_ref, p_ref, tt_ref, ty_ref, w_ref, b_ref, o_ref):
    x = g_ref[0] + p_ref[...]
    t0 = ty_ref[0:1, :]
    t1 = ty_ref[1:2, :]
    tt = tt_ref[0].astype(jnp.float32)
    x = x + t0 + tt * (t1 - t0)
    mean = jnp.mean(x, axis=1, keepdims=True)
    xc = x - mean
    var = jnp.mean(xc * xc, axis=1, keepdims=True)
    y = xc * lax.rsqrt(var + _EPS)
    o_ref[0] = y * w_ref[...] + b_ref[...]


def _ln_body_acc(g_ref, p_ref, tt_ref, ty_ref, w_ref, b_ref, _buf_ref, o_ref):
    _ln_body(g_ref, p_ref, tt_ref, ty_ref, w_ref, b_ref, o_ref)


def _tc_add_ln_chunk(g_c, pos_table, tt3, type_table, ln_w, ln_b, buf, c, b, s):
    s_ch = g_c.shape[1]
    sblk0 = c * (s_ch // _TBLK)
    grid = (s_ch // _TBLK, b)
    in_specs = [
        pl.BlockSpec((1, _TBLK, _HIDDEN), lambda i, j: (j, i, 0)),
        pl.BlockSpec((_TBLK, _HIDDEN), lambda i, j, c0=sblk0: (c0 + i, 0)),
        pl.BlockSpec((1, _TBLK, 1), lambda i, j, c0=sblk0: (j, c0 + i, 0)),
        pl.BlockSpec((2, _HIDDEN), lambda i, j: (0, 0)),
        pl.BlockSpec((1, _HIDDEN), lambda i, j: (0, 0)),
        pl.BlockSpec((1, _HIDDEN), lambda i, j: (0, 0)),
    ]
    args = [g_c, pos_table, tt3, type_table, ln_w, ln_b]
    body = _ln_body
    aliases = {}
    if buf is not None:
        in_specs.append(pl.BlockSpec(memory_space=pltpu.MemorySpace.HBM))
        args.append(buf)
        aliases = {6: 0}
        body = _ln_body_acc
    return pl.pallas_call(
        body,
        grid=grid,
        in_specs=in_specs,
        out_specs=pl.BlockSpec((1, _TBLK, _HIDDEN), lambda i, j, c0=sblk0: (j, c0 + i, 0)),
        out_shape=jax.ShapeDtypeStruct((b, s, _HIDDEN), jnp.float32),
        input_output_aliases=aliases,
    )(*args)


def kernel(input_ids, token_type_ids, word_table, pos_table, type_table, ln_weight, ln_bias):
    b, s = input_ids.shape
    s_ch = s // _NCHUNK
    ln_w = ln_weight.reshape(1, -1)
    ln_b = ln_bias.reshape(1, -1)
    idx_flat = input_ids.reshape(-1).astype(jnp.int32)
    tt3 = token_type_ids.reshape(b, s, 1)

    gathered = []
    for c in range(_NCHUNK):
        g_c = _sc_gather_chunk(word_table, idx_flat, c * s_ch, b * s_ch, s, s_ch)
        gathered.append(g_c.reshape(b, s_ch, _HIDDEN))

    buf = None
    for c in range(_NCHUNK):
        buf = _tc_add_ln_chunk(
            gathered[c], pos_table, tt3, type_table, ln_w, ln_b, buf, c, b, s
        )
    return buf


# speedup vs baseline: 2.3197x; 1.0470x over previous
import functools

import jax
import jax.numpy as jnp
from jax import lax
from jax.experimental import pallas as pl
from jax.experimental.pallas import tpu as pltpu
from jax.experimental.pallas import tpu_sc as plsc

_HIDDEN = 768
_EPS = 1e-12
_NC = 2
_NS = 16
_NW = _NC * _NS
_CHUNK = 32
_NCHUNK = 4
_TBLK = 512


def _sc_gather_chunk(word_table, idx_flat, chunk_base, n_rows, s, s_ch):
    per_w = n_rows // _NW
    n_sub = per_w // _CHUNK
    w_per_b = s_ch // per_w
    mesh = plsc.VectorSubcoreMesh(core_axis_name="c", subcore_axis_name="s")

    @functools.partial(
        pl.kernel,
        out_type=jax.ShapeDtypeStruct((n_rows, _HIDDEN), jnp.float32),
        mesh=mesh,
        scratch_types=[
            pltpu.VMEM((per_w,), jnp.int32),
            pltpu.VMEM((_CHUNK, _HIDDEN), jnp.float32),
            pltpu.VMEM((_CHUNK, _HIDDEN), jnp.float32),
            pltpu.SemaphoreType.DMA,
            pltpu.SemaphoreType.DMA,
        ],
    )
    def gather_kernel(table_hbm, idx_hbm, out_hbm, idx_v, rows_a, rows_b, sem_a, sem_b):
        wid = lax.axis_index("s") * _NC + lax.axis_index("c")
        b = wid // w_per_b
        j = (wid % w_per_b) * per_w
        src = b * s + chunk_base + j
        pltpu.sync_copy(idx_hbm.at[pl.ds(src, per_w)], idx_v)

        base = wid * per_w
        bufs = (rows_a, rows_b)
        sems = (sem_a, sem_b)
        copies = [None] * n_sub
        copies[0] = pltpu.async_copy(
            table_hbm.at[idx_v.at[pl.ds(0, _CHUNK)]], bufs[0], sems[0]
        )
        for c in range(n_sub):
            if c + 1 < n_sub:
                copies[c + 1] = pltpu.async_copy(
                    table_hbm.at[idx_v.at[pl.ds((c + 1) * _CHUNK, _CHUNK)]],
                    bufs[(c + 1) % 2],
                    sems[(c + 1) % 2],
                )
            copies[c].wait()
            pltpu.sync_copy(bufs[c % 2], out_hbm.at[pl.ds(base + c * _CHUNK, _CHUNK)])

    return gather_kernel(word_table, idx_flat)


def _ln_body(g_ref, p_ref, tt_ref, ty_ref, w_ref, b_ref, o_ref):
    x = g_ref[0] + p_ref[...]
    t0 = ty_ref[0:1, :]
    t1 = ty_ref[1:2, :]
    tt = tt_ref[0].astype(jnp.float32)
    x = x + t0 + tt * (t1 - t0)
    mean = jnp.mean(x, axis=1, keepdims=True)
    xc = x - mean
    var = jnp.mean(xc * xc, axis=1, keepdims=True)
    y = xc * lax.rsqrt(var + _EPS)
    o_ref[0] = y * w_ref[...] + b_ref[...]


def _ln_body_acc(g_ref, p_ref, tt_ref, ty_ref, w_ref, b_ref, _buf_ref, o_ref):
    _ln_body(g_ref, p_ref, tt_ref, ty_ref, w_ref, b_ref, o_ref)


def _tc_add_ln_chunk(g_c, pos_table, tt3, type_table, ln_w, ln_b, buf, c, b, s):
    s_ch = g_c.shape[1]
    sblk0 = c * (s_ch // _TBLK)
    grid = (s_ch // _TBLK, b)
    in_specs = [
        pl.BlockSpec((1, _TBLK, _HIDDEN), lambda i, j: (j, i, 0)),
        pl.BlockSpec((_TBLK, _HIDDEN), lambda i, j, c0=sblk0: (c0 + i, 0)),
        pl.BlockSpec((1, _TBLK, 1), lambda i, j, c0=sblk0: (j, c0 + i, 0)),
        pl.BlockSpec((2, _HIDDEN), lambda i, j: (0, 0)),
        pl.BlockSpec((1, _HIDDEN), lambda i, j: (0, 0)),
        pl.BlockSpec((1, _HIDDEN), lambda i, j: (0, 0)),
    ]
    args = [g_c, pos_table, tt3, type_table, ln_w, ln_b]
    body = _ln_body
    aliases = {}
    if buf is not None:
        in_specs.append(pl.BlockSpec(memory_space=pltpu.MemorySpace.HBM))
        args.append(buf)
        aliases = {6: 0}
        body = _ln_body_acc
    return pl.pallas_call(
        body,
        grid=grid,
        in_specs=in_specs,
        out_specs=pl.BlockSpec((1, _TBLK, _HIDDEN), lambda i, j, c0=sblk0: (j, c0 + i, 0)),
        out_shape=jax.ShapeDtypeStruct((b, s, _HIDDEN), jnp.float32),
        input_output_aliases=aliases,
    )(*args)


def kernel(input_ids, token_type_ids, word_table, pos_table, type_table, ln_weight, ln_bias):
    b, s = input_ids.shape
    s_ch = s // _NCHUNK
    ln_w = ln_weight.reshape(1, -1)
    ln_b = ln_bias.reshape(1, -1)
    idx_flat = input_ids.reshape(-1).astype(jnp.int32)
    tt3 = token_type_ids.reshape(b, s, 1)

    gathered = []
    for c in range(_NCHUNK):
        g_c = _sc_gather_chunk(word_table, idx_flat, c * s_ch, b * s_ch, s, s_ch)
        gathered.append(g_c.reshape(b, s_ch, _HIDDEN))

    buf = None
    for c in range(_NCHUNK):
        buf = _tc_add_ln_chunk(
            gathered[c], pos_table, tt3, type_table, ln_w, ln_b, buf, c, b, s
        )
    return buf


# speedup vs baseline: 2.3719x; 1.0225x over previous
import functools

import jax
import jax.numpy as jnp
from jax import lax
from jax.experimental import pallas as pl
from jax.experimental.pallas import tpu as pltpu
from jax.experimental.pallas import tpu_sc as plsc

_HIDDEN = 768
_EPS = 1e-12
_NC = 2
_NS = 16
_NW = _NC * _NS
_CHUNK = 64
_NSC = 2
_NTC = 4
_TBLK = 512


def _sc_gather_chunk(word_table, idx_flat, chunk_base, n_rows, s, s_ch):
    per_w = n_rows // _NW
    n_sub = per_w // _CHUNK
    w_per_b = s_ch // per_w
    mesh = plsc.VectorSubcoreMesh(core_axis_name="c", subcore_axis_name="s")

    @functools.partial(
        pl.kernel,
        out_type=jax.ShapeDtypeStruct((n_rows, _HIDDEN), jnp.float32),
        mesh=mesh,
        scratch_types=[
            pltpu.VMEM((per_w,), jnp.int32),
            pltpu.VMEM((_CHUNK, _HIDDEN), jnp.float32),
            pltpu.VMEM((_CHUNK, _HIDDEN), jnp.float32),
            pltpu.SemaphoreType.DMA,
            pltpu.SemaphoreType.DMA,
        ],
    )
    def gather_kernel(table_hbm, idx_hbm, out_hbm, idx_v, rows_a, rows_b, sem_a, sem_b):
        wid = lax.axis_index("s") * _NC + lax.axis_index("c")
        b = wid // w_per_b
        j = (wid % w_per_b) * per_w
        src = b * s + chunk_base + j
        pltpu.sync_copy(idx_hbm.at[pl.ds(src, per_w)], idx_v)

        base = wid * per_w
        bufs = (rows_a, rows_b)
        sems = (sem_a, sem_b)
        copies = [None] * n_sub
        copies[0] = pltpu.async_copy(
            table_hbm.at[idx_v.at[pl.ds(0, _CHUNK)]], bufs[0], sems[0]
        )
        for c in range(n_sub):
            if c + 1 < n_sub:
                copies[c + 1] = pltpu.async_copy(
                    table_hbm.at[idx_v.at[pl.ds((c + 1) * _CHUNK, _CHUNK)]],
                    bufs[(c + 1) % 2],
                    sems[(c + 1) % 2],
                )
            copies[c].wait()
            pltpu.sync_copy(bufs[c % 2], out_hbm.at[pl.ds(base + c * _CHUNK, _CHUNK)])

    return gather_kernel(word_table, idx_flat)


def _ln_body(g_ref, p_ref, tt_ref, ty_ref, w_ref, b_ref, o_ref):
    x = g_ref[0] + p_ref[...]
    t0 = ty_ref[0:1, :]
    t1 = ty_ref[1:2, :]
    tt = tt_ref[0].astype(jnp.float32)
    x = x + t0 + tt * (t1 - t0)
    mean = jnp.mean(x, axis=1, keepdims=True)
    xc = x - mean
    var = jnp.mean(xc * xc, axis=1, keepdims=True)
    y = xc * lax.rsqrt(var + _EPS)
    o_ref[0] = y * w_ref[...] + b_ref[...]


def _ln_body_acc(g_ref, p_ref, tt_ref, ty_ref, w_ref, b_ref, _buf_ref, o_ref):
    _ln_body(g_ref, p_ref, tt_ref, ty_ref, w_ref, b_ref, o_ref)


def _tc_add_ln_chunk(g_h, pos_table, tt3, type_table, ln_w, ln_b, buf, c, b, s):
    s_q = s // _NTC
    n_blk = s_q // _TBLK
    qin = (c % (_NTC // _NSC)) * n_blk
    qout = c * n_blk
    grid = (n_blk, b)
    in_specs = [
        pl.BlockSpec((1, _TBLK, _HIDDEN), lambda i, j, q=qin: (j, q + i, 0)),
        pl.BlockSpec((_TBLK, _HIDDEN), lambda i, j, q=qout: (q + i, 0)),
        pl.BlockSpec((1, _TBLK, 1), lambda i, j, q=qout: (j, q + i, 0)),
        pl.BlockSpec((2, _HIDDEN), lambda i, j: (0, 0)),
        pl.BlockSpec((1, _HIDDEN), lambda i, j: (0, 0)),
        pl.BlockSpec((1, _HIDDEN), lambda i, j: (0, 0)),
    ]
    args = [g_h, pos_table, tt3, type_table, ln_w, ln_b]
    body = _ln_body
    aliases = {}
    if buf is not None:
        in_specs.append(pl.BlockSpec(memory_space=pltpu.MemorySpace.HBM))
        args.append(buf)
        aliases = {6: 0}
        body = _ln_body_acc
    return pl.pallas_call(
        body,
        grid=grid,
        in_specs=in_specs,
        out_specs=pl.BlockSpec((1, _TBLK, _HIDDEN), lambda i, j, q=qout: (j, q + i, 0)),
        out_shape=jax.ShapeDtypeStruct((b, s, _HIDDEN), jnp.float32),
        input_output_aliases=aliases,
    )(*args)


def kernel(input_ids, token_type_ids, word_table, pos_table, type_table, ln_weight, ln_bias):
    b, s = input_ids.shape
    s_h = s // _NSC
    ln_w = ln_weight.reshape(1, -1)
    ln_b = ln_bias.reshape(1, -1)
    idx_flat = input_ids.reshape(-1).astype(jnp.int32)
    tt3 = token_type_ids.reshape(b, s, 1)

    gathered = []
    for h in range(_NSC):
        g_h = _sc_gather_chunk(word_table, idx_flat, h * s_h, b * s_h, s, s_h)
        gathered.append(g_h.reshape(b, s_h, _HIDDEN))

    buf = None
    for c in range(_NTC):
        buf = _tc_add_ln_chunk(
            gathered[c // (_NTC // _NSC)], pos_table, tt3, type_table,
            ln_w, ln_b, buf, c, b, s,
        )
    return buf
